# baseline (speedup 1.0000x reference)
import math

import jax
import jax.numpy as jnp
from jax import lax
from jax.experimental import pallas as pl
from jax.experimental.pallas import tpu as pltpu

_H = 16
_E = 64
_BKT = 128
_NHASH = 4


def _attn_body(q_ref, k_ref, v_ref, o_ref, l_ref):
    temp = 1.0 / math.sqrt(_E)
    qb = q_ref[0]
    kb = k_ref[0]
    vb = v_ref[0]
    s = lax.dot_general(qb, kb, (((1,), (1,)), ((), ())),
                        preferred_element_type=jnp.float32) * temp
    m = jnp.max(s, axis=-1, keepdims=True)
    p = jnp.exp(s - m)
    den = jnp.sum(p, axis=-1, keepdims=True)
    o = lax.dot_general(p, vb, (((1,), (0,)), ((), ())),
                        preferred_element_type=jnp.float32)
    o_ref[0] = o / den
    l_ref[0] = m + jnp.log(den)


def _combine_body(o_ref, l_ref, w_ref, b_ref, out_ref):
    acc = jnp.zeros((_BKT, _E), jnp.float32)
    for h in range(_H):
        lh = l_ref[:, 0, h, :, :]
        mh = jnp.max(lh, axis=0, keepdims=True)
        wgt = jnp.exp(lh - mh)
        den = jnp.sum(wgt, axis=0)
        oh = o_ref[:, 0, h, :, :]
        num = jnp.sum(oh * wgt, axis=0)
        ch = num / den
        wh = w_ref[:, h * _E:(h + 1) * _E]
        acc = acc + lax.dot_general(ch, wh, (((1,), (1,)), ((), ())),
                                    preferred_element_type=jnp.float32)
    out_ref[0] = acc + b_ref[0]


def kernel(query, key, value, alpha, beta, W_out, b_out):
    B, N, D = query.shape
    H, E = _H, _E
    BH = B * H
    nbkt = N // _BKT

    q = query.reshape(B, N, H, E).transpose(0, 2, 1, 3).reshape(BH, N, E)
    k = key.reshape(B, N, H, E).transpose(0, 2, 1, 3).reshape(BH, N, E)
    v = value.reshape(B, N, H, E).transpose(0, 2, 1, 3).reshape(BH, N, E)

    q_norms = jnp.linalg.norm(q, axis=-1, keepdims=True)
    k_norms = jnp.linalg.norm(k, axis=-1, keepdims=True)
    MQ = jnp.max(q_norms, axis=-2, keepdims=True)
    MK = jnp.max(k_norms, axis=-2, keepdims=True)
    q_ext = jnp.sqrt(jnp.maximum(MQ ** 2 + MK ** 2 - q_norms ** 2, 0.0))
    k_ext = jnp.sqrt(jnp.maximum(MQ ** 2 + MK ** 2 - k_norms ** 2, 0.0))
    q_t = jnp.concatenate([q, q_ext, jnp.zeros_like(q_ext)], axis=-1)
    k_t = jnp.concatenate([k, jnp.zeros_like(k_ext), k_ext], axis=-1)
    q_hash = jnp.transpose(q_t @ alpha + beta, (2, 0, 1))
    k_hash = jnp.transpose(k_t @ alpha + beta, (2, 0, 1))

    pq = jnp.argsort(q_hash, axis=-1)
    pk = jnp.argsort(k_hash, axis=-1)

    def bucket_gather(x, perm):
        xb = jnp.broadcast_to(x[None], (_NHASH, BH, N, E))
        return jnp.take_along_axis(xb, perm[..., None], axis=2)

    s_q = bucket_gather(q, pq).reshape(_NHASH * BH, N, E)
    s_k = bucket_gather(k, pk).reshape(_NHASH * BH, N, E)
    s_v = bucket_gather(v, pk).reshape(_NHASH * BH, N, E)

    G = _NHASH * BH
    o_s, l_s = pl.pallas_call(
        _attn_body,
        grid=(G, nbkt),
        in_specs=[
            pl.BlockSpec((1, _BKT, E), lambda g, b: (g, b, 0)),
            pl.BlockSpec((1, _BKT, E), lambda g, b: (g, b, 0)),
            pl.BlockSpec((1, _BKT, E), lambda g, b: (g, b, 0)),
        ],
        out_specs=[
            pl.BlockSpec((1, _BKT, E), lambda g, b: (g, b, 0)),
            pl.BlockSpec((1, _BKT, 1), lambda g, b: (g, b, 0)),
        ],
        out_shape=[
            jax.ShapeDtypeStruct((G, N, E), jnp.float32),
            jax.ShapeDtypeStruct((G, N, 1), jnp.float32),
        ],
    )(s_q, s_k, s_v)

    o_su = o_s.reshape(_NHASH, BH, N, E)
    l_su = l_s.reshape(_NHASH, BH, N, 1)
    pq_b = jnp.broadcast_to(pq[..., None], o_su.shape)
    o_u = jnp.put_along_axis(jnp.empty_like(o_su), pq_b, o_su,
                             axis=2, inplace=False)
    l_u = jnp.put_along_axis(jnp.empty_like(l_su), pq[..., None], l_su,
                             axis=2, inplace=False)

    o_u = o_u.reshape(_NHASH, B, H, N, E)
    l_u = l_u.reshape(_NHASH, B, H, N, 1)

    out = pl.pallas_call(
        _combine_body,
        grid=(B, nbkt),
        in_specs=[
            pl.BlockSpec((_NHASH, 1, H, _BKT, E), lambda b, n: (0, b, 0, n, 0)),
            pl.BlockSpec((_NHASH, 1, H, _BKT, 1), lambda b, n: (0, b, 0, n, 0)),
            pl.BlockSpec((E, H * E), lambda b, n: (0, 0)),
            pl.BlockSpec((1, E), lambda b, n: (0, 0)),
        ],
        out_specs=pl.BlockSpec((1, _BKT, E), lambda b, n: (b, n, 0)),
        out_shape=jax.ShapeDtypeStruct((B, N, E), jnp.float32),
    )(o_u, l_u, W_out, b_out.reshape(1, E))
    return out


# speedup vs baseline: 26.4778x; 26.4778x over previous
import math

import jax
import jax.numpy as jnp
from jax import lax
from jax.experimental import pallas as pl
from jax.experimental.pallas import tpu as pltpu

_H = 16
_E = 64
_BKT = 128
_NHASH = 4


def _attn_body(q_ref, k_ref, v_ref, o_ref, l_ref):
    temp = 1.0 / math.sqrt(_E)
    qb = q_ref[0]
    kb = k_ref[0]
    vb = v_ref[0]
    s = lax.dot_general(qb, kb, (((1,), (1,)), ((), ())),
                        preferred_element_type=jnp.float32) * temp
    m = jnp.max(s, axis=-1, keepdims=True)
    p = jnp.exp(s - m)
    den = jnp.sum(p, axis=-1, keepdims=True)
    o = lax.dot_general(p, vb, (((1,), (0,)), ((), ())),
                        preferred_element_type=jnp.float32)
    o_ref[0] = o / den
    l_ref[0] = m + jnp.log(den)


def _combine_body(o_ref, l_ref, w_ref, b_ref, out_ref):
    acc = jnp.zeros((_BKT, _E), jnp.float32)
    for h in range(_H):
        lh = l_ref[:, 0, h, :, :]
        mh = jnp.max(lh, axis=0, keepdims=True)
        wgt = jnp.exp(lh - mh)
        den = jnp.sum(wgt, axis=0)
        oh = o_ref[:, 0, h, :, :]
        num = jnp.sum(oh * wgt, axis=0)
        ch = num / den
        wh = w_ref[:, h * _E:(h + 1) * _E]
        acc = acc + lax.dot_general(ch, wh, (((1,), (1,)), ((), ())),
                                    preferred_element_type=jnp.float32)
    out_ref[0] = acc + b_ref[0]


def kernel(query, key, value, alpha, beta, W_out, b_out):
    B, N, D = query.shape
    H, E = _H, _E
    BH = B * H
    nbkt = N // _BKT

    q = query.reshape(B, N, H, E).transpose(0, 2, 1, 3).reshape(BH, N, E)
    k = key.reshape(B, N, H, E).transpose(0, 2, 1, 3).reshape(BH, N, E)
    v = value.reshape(B, N, H, E).transpose(0, 2, 1, 3).reshape(BH, N, E)

    q_norms = jnp.linalg.norm(q, axis=-1, keepdims=True)
    k_norms = jnp.linalg.norm(k, axis=-1, keepdims=True)
    MQ = jnp.max(q_norms, axis=-2, keepdims=True)
    MK = jnp.max(k_norms, axis=-2, keepdims=True)
    q_ext = jnp.sqrt(jnp.maximum(MQ ** 2 + MK ** 2 - q_norms ** 2, 0.0))
    k_ext = jnp.sqrt(jnp.maximum(MQ ** 2 + MK ** 2 - k_norms ** 2, 0.0))
    q_t = jnp.concatenate([q, q_ext, jnp.zeros_like(q_ext)], axis=-1)
    k_t = jnp.concatenate([k, jnp.zeros_like(k_ext), k_ext], axis=-1)
    q_hash = jnp.transpose(q_t @ alpha + beta, (2, 0, 1))
    k_hash = jnp.transpose(k_t @ alpha + beta, (2, 0, 1))

    pq = jnp.argsort(q_hash, axis=-1)
    pk = jnp.argsort(k_hash, axis=-1)

    def bucket_gather(x, perm):
        xb = jnp.broadcast_to(x[None], (_NHASH, BH, N, E))
        return jnp.take_along_axis(xb, perm[..., None], axis=2)

    s_q = bucket_gather(q, pq).reshape(_NHASH * BH, N, E)
    s_k = bucket_gather(k, pk).reshape(_NHASH * BH, N, E)
    s_v = bucket_gather(v, pk).reshape(_NHASH * BH, N, E)

    G = _NHASH * BH
    o_s, l_s = pl.pallas_call(
        _attn_body,
        grid=(G, nbkt),
        in_specs=[
            pl.BlockSpec((1, _BKT, E), lambda g, b: (g, b, 0)),
            pl.BlockSpec((1, _BKT, E), lambda g, b: (g, b, 0)),
            pl.BlockSpec((1, _BKT, E), lambda g, b: (g, b, 0)),
        ],
        out_specs=[
            pl.BlockSpec((1, _BKT, E), lambda g, b: (g, b, 0)),
            pl.BlockSpec((1, _BKT, 1), lambda g, b: (g, b, 0)),
        ],
        out_shape=[
            jax.ShapeDtypeStruct((G, N, E), jnp.float32),
            jax.ShapeDtypeStruct((G, N, 1), jnp.float32),
        ],
    )(s_q, s_k, s_v)

    o_su = o_s.reshape(_NHASH, BH, N, E)
    l_su = l_s.reshape(_NHASH, BH, N, 1)
    pq_inv = jnp.argsort(pq, axis=-1)
    o_u = jnp.take_along_axis(o_su, pq_inv[..., None], axis=2)
    l_u = jnp.take_along_axis(l_su, pq_inv[..., None], axis=2)

    o_u = o_u.reshape(_NHASH, B, H, N, E)
    l_u = l_u.reshape(_NHASH, B, H, N, 1)

    out = pl.pallas_call(
        _combine_body,
        grid=(B, nbkt),
        in_specs=[
            pl.BlockSpec((_NHASH, 1, H, _BKT, E), lambda b, n: (0, b, 0, n, 0)),
            pl.BlockSpec((_NHASH, 1, H, _BKT, 1), lambda b, n: (0, b, 0, n, 0)),
            pl.BlockSpec((E, H * E), lambda b, n: (0, 0)),
            pl.BlockSpec((1, E), lambda b, n: (0, 0)),
        ],
        out_specs=pl.BlockSpec((1, _BKT, E), lambda b, n: (b, n, 0)),
        out_shape=jax.ShapeDtypeStruct((B, N, E), jnp.float32),
    )(o_u, l_u, W_out, b_out.reshape(1, E))
    return out


# speedup vs baseline: 37.1467x; 1.4029x over previous
import math

import jax
import jax.numpy as jnp
from jax import lax
from jax.experimental import pallas as pl
from jax.experimental.pallas import tpu as pltpu

_H = 16
_E = 64
_BKT = 128
_NHASH = 4


_BKT_PER_STEP = 8


def _attn_body(q_ref, k_ref, v_ref, o_ref, l_ref):
    temp = 1.0 / math.sqrt(_E)
    for i in range(_BKT_PER_STEP):
        sl = pl.ds(i * _BKT, _BKT)
        qb = q_ref[0, sl, :]
        kb = k_ref[0, sl, :]
        vb = v_ref[0, sl, :]
        s = lax.dot_general(qb, kb, (((1,), (1,)), ((), ())),
                            preferred_element_type=jnp.float32) * temp
        m = jnp.max(s, axis=-1, keepdims=True)
        p = jnp.exp(s - m)
        den = jnp.sum(p, axis=-1, keepdims=True)
        o = lax.dot_general(p, vb, (((1,), (0,)), ((), ())),
                            preferred_element_type=jnp.float32)
        o_ref[0, sl, :] = o / den
        l_ref[0, sl, :] = m + jnp.log(den)


def _combine_body(o_ref, l_ref, w_ref, b_ref, out_ref):
    acc = jnp.zeros((_BKT, _E), jnp.float32)
    for h in range(_H):
        lh = l_ref[:, 0, h, :, :]
        mh = jnp.max(lh, axis=0, keepdims=True)
        wgt = jnp.exp(lh - mh)
        den = jnp.sum(wgt, axis=0)
        oh = o_ref[:, 0, h, :, :]
        num = jnp.sum(oh * wgt, axis=0)
        ch = num / den
        wh = w_ref[:, h * _E:(h + 1) * _E]
        acc = acc + lax.dot_general(ch, wh, (((1,), (1,)), ((), ())),
                                    preferred_element_type=jnp.float32)
    out_ref[0] = acc + b_ref[0]


def kernel(query, key, value, alpha, beta, W_out, b_out):
    B, N, D = query.shape
    H, E = _H, _E
    BH = B * H
    nbkt = N // _BKT

    q = query.reshape(B, N, H, E).transpose(0, 2, 1, 3).reshape(BH, N, E)
    k = key.reshape(B, N, H, E).transpose(0, 2, 1, 3).reshape(BH, N, E)
    v = value.reshape(B, N, H, E).transpose(0, 2, 1, 3).reshape(BH, N, E)

    q_norms = jnp.linalg.norm(q, axis=-1, keepdims=True)
    k_norms = jnp.linalg.norm(k, axis=-1, keepdims=True)
    MQ = jnp.max(q_norms, axis=-2, keepdims=True)
    MK = jnp.max(k_norms, axis=-2, keepdims=True)
    q_ext = jnp.sqrt(jnp.maximum(MQ ** 2 + MK ** 2 - q_norms ** 2, 0.0))
    k_ext = jnp.sqrt(jnp.maximum(MQ ** 2 + MK ** 2 - k_norms ** 2, 0.0))
    q_t = jnp.concatenate([q, q_ext, jnp.zeros_like(q_ext)], axis=-1)
    k_t = jnp.concatenate([k, jnp.zeros_like(k_ext), k_ext], axis=-1)
    q_hash = jnp.transpose(q_t @ alpha + beta, (2, 0, 1))
    k_hash = jnp.transpose(k_t @ alpha + beta, (2, 0, 1))

    pq = jnp.argsort(q_hash, axis=-1)
    pk = jnp.argsort(k_hash, axis=-1)

    def bucket_gather(x, perm):
        xb = jnp.broadcast_to(x[None], (_NHASH, BH, N, E))
        return jnp.take_along_axis(xb, perm[..., None], axis=2)

    s_q = bucket_gather(q, pq).reshape(_NHASH * BH, N, E)
    s_k = bucket_gather(k, pk).reshape(_NHASH * BH, N, E)
    s_v = bucket_gather(v, pk).reshape(_NHASH * BH, N, E)

    G = _NHASH * BH
    W = _BKT_PER_STEP * _BKT
    o_s, l_s = pl.pallas_call(
        _attn_body,
        grid=(G, N // W),
        in_specs=[
            pl.BlockSpec((1, W, E), lambda g, b: (g, b, 0)),
            pl.BlockSpec((1, W, E), lambda g, b: (g, b, 0)),
            pl.BlockSpec((1, W, E), lambda g, b: (g, b, 0)),
        ],
        out_specs=[
            pl.BlockSpec((1, W, E), lambda g, b: (g, b, 0)),
            pl.BlockSpec((1, W, 1), lambda g, b: (g, b, 0)),
        ],
        out_shape=[
            jax.ShapeDtypeStruct((G, N, E), jnp.float32),
            jax.ShapeDtypeStruct((G, N, 1), jnp.float32),
        ],
    )(s_q, s_k, s_v)

    o_su = o_s.reshape(_NHASH, BH, N, E)
    l_su = l_s.reshape(_NHASH, BH, N, 1)
    pq_inv = jnp.argsort(pq, axis=-1)
    o_u = jnp.take_along_axis(o_su, pq_inv[..., None], axis=2)
    l_u = jnp.take_along_axis(l_su, pq_inv[..., None], axis=2)

    o_u = o_u.reshape(_NHASH, B, H, N, E)
    l_u = l_u.reshape(_NHASH, B, H, N, 1)

    out = pl.pallas_call(
        _combine_body,
        grid=(B, nbkt),
        in_specs=[
            pl.BlockSpec((_NHASH, 1, H, _BKT, E), lambda b, n: (0, b, 0, n, 0)),
            pl.BlockSpec((_NHASH, 1, H, _BKT, 1), lambda b, n: (0, b, 0, n, 0)),
            pl.BlockSpec((E, H * E), lambda b, n: (0, 0)),
            pl.BlockSpec((1, E), lambda b, n: (0, 0)),
        ],
        out_specs=pl.BlockSpec((1, _BKT, E), lambda b, n: (b, n, 0)),
        out_shape=jax.ShapeDtypeStruct((B, N, E), jnp.float32),
    )(o_u, l_u, W_out, b_out.reshape(1, E))
    return out


# speedup vs baseline: 53.9166x; 1.4514x over previous
import math

import jax
import jax.numpy as jnp
from jax import lax
from jax.experimental import pallas as pl
from jax.experimental.pallas import tpu as pltpu
from jax.experimental.pallas import tpu_sc as plsc

_H = 16
_E = 64
_BKT = 128
_NHASH = 4

_NW = 32
_CHUNK = 128
_ROW = 2 * _E
_BKT_PER_STEP = 8


def _attn_body(q_ref, kv_ref, ol_ref):
    temp = 1.0 / math.sqrt(_E)
    for i in range(_BKT_PER_STEP):
        sl = pl.ds(i * _BKT, _BKT)
        qb = q_ref[0, sl, 0:_E]
        kb = kv_ref[0, sl, 0:_E]
        vb = kv_ref[0, sl, _E:_ROW]
        s = lax.dot_general(qb, kb, (((1,), (1,)), ((), ())),
                            preferred_element_type=jnp.float32) * temp
        m = jnp.max(s, axis=-1, keepdims=True)
        p = jnp.exp(s - m)
        den = jnp.sum(p, axis=-1, keepdims=True)
        o = lax.dot_general(p, vb, (((1,), (0,)), ((), ())),
                            preferred_element_type=jnp.float32)
        lse = jnp.broadcast_to(m + jnp.log(den), (_BKT, _E))
        ol_ref[0, sl, :] = jnp.concatenate([o / den, lse], axis=-1)


def _combine_body(ol_ref, w_ref, b_ref, out_ref):
    acc = jnp.zeros((_BKT, _E), jnp.float32)
    for h in range(_H):
        lh = ol_ref[:, 0, h, :, _E:_E + 1]
        mh = jnp.max(lh, axis=0, keepdims=True)
        wgt = jnp.exp(lh - mh)
        den = jnp.sum(wgt, axis=0)
        oh = ol_ref[:, 0, h, :, 0:_E]
        num = jnp.sum(oh * wgt, axis=0)
        ch = num / den
        wh = w_ref[:, h * _E:(h + 1) * _E]
        acc = acc + lax.dot_general(ch, wh, (((1,), (1,)), ((), ())),
                                    preferred_element_type=jnp.float32)
    out_ref[0] = acc + b_ref[0]


def _sc_gather_body(qq_hbm, kv_hbm, iq_hbm, ik_hbm,
                    sq_hbm, skv_hbm,
                    iq_v, ik_v, bq, bkv, sem):
    wid = lax.axis_index("s") * 2 + lax.axis_index("c")
    nchunk = iq_hbm.shape[0] // _NW
    pltpu.sync_copy(iq_hbm.at[pl.ds(wid * nchunk, nchunk)], iq_v)
    pltpu.sync_copy(ik_hbm.at[pl.ds(wid * nchunk, nchunk)], ik_v)
    for j in range(64):
        row0 = (wid * 64 + j) * _CHUNK
        cq = pltpu.async_copy(qq_hbm.at[iq_v.at[j]], bq, sem)
        ck = pltpu.async_copy(kv_hbm.at[ik_v.at[j]], bkv, sem)
        cq.wait()
        ck.wait()
        pltpu.sync_copy(bq, sq_hbm.at[pl.ds(row0, _CHUNK)])
        pltpu.sync_copy(bkv, skv_hbm.at[pl.ds(row0, _CHUNK)])


def _sc_scatter_body(ol_hbm, si_hbm, ou_hbm, si_v, bo, sem):
    wid = lax.axis_index("s") * 2 + lax.axis_index("c")
    nchunk = si_hbm.shape[0] // _NW
    pltpu.sync_copy(si_hbm.at[pl.ds(wid * nchunk, nchunk)], si_v)
    for j in range(64):
        row0 = (wid * 64 + j) * _CHUNK
        pltpu.sync_copy(ol_hbm.at[pl.ds(row0, _CHUNK)], bo)
        pltpu.async_copy(bo, ou_hbm.at[si_v.at[j]], sem).wait()


def kernel(query, key, value, alpha, beta, W_out, b_out):
    B, N, D = query.shape
    H, E = _H, _E
    BH = B * H
    nbkt = N // _BKT
    R = _NHASH * BH * N

    q = query.reshape(B, N, H, E).transpose(0, 2, 1, 3).reshape(BH, N, E)
    k = key.reshape(B, N, H, E).transpose(0, 2, 1, 3).reshape(BH, N, E)

    q_norms = jnp.linalg.norm(q, axis=-1, keepdims=True)
    k_norms = jnp.linalg.norm(k, axis=-1, keepdims=True)
    MQ = jnp.max(q_norms, axis=-2, keepdims=True)
    MK = jnp.max(k_norms, axis=-2, keepdims=True)
    q_ext = jnp.sqrt(jnp.maximum(MQ ** 2 + MK ** 2 - q_norms ** 2, 0.0))
    k_ext = jnp.sqrt(jnp.maximum(MQ ** 2 + MK ** 2 - k_norms ** 2, 0.0))
    q_t = jnp.concatenate([q, q_ext, jnp.zeros_like(q_ext)], axis=-1)
    k_t = jnp.concatenate([k, jnp.zeros_like(k_ext), k_ext], axis=-1)
    q_hash = jnp.transpose(q_t @ alpha + beta, (2, 0, 1))
    k_hash = jnp.transpose(k_t @ alpha + beta, (2, 0, 1))

    pq = jnp.argsort(q_hash, axis=-1)
    pk = jnp.argsort(k_hash, axis=-1)

    bh = jnp.arange(BH, dtype=jnp.int32)
    b_of = (bh // H)[None, :, None]
    h_of = (bh % H)[None, :, None]
    iq = ((b_of * N + pq) * H + h_of).reshape(R // _CHUNK, _CHUNK)
    ik = ((b_of * N + pk) * H + h_of).reshape(R // _CHUNK, _CHUNK)
    si = (pq + (jnp.arange(_NHASH * BH, dtype=jnp.int32) * N)
          .reshape(_NHASH, BH, 1)).reshape(R // _CHUNK, _CHUNK)

    q4 = query.reshape(B, N, H, E)
    qq = jnp.concatenate([q4, q4], axis=-1).reshape(B * N * H, _ROW)
    kv = jnp.concatenate([key.reshape(B, N, H, E),
                          value.reshape(B, N, H, E)],
                         axis=-1).reshape(B * N * H, _ROW)

    mesh = plsc.VectorSubcoreMesh(core_axis_name="c", subcore_axis_name="s")
    s_q, s_kv = pl.kernel(
        _sc_gather_body,
        mesh=mesh,
        out_type=[jax.ShapeDtypeStruct((R, _ROW), jnp.float32)] * 2,
        scratch_types=[
            pltpu.VMEM((R // _CHUNK // _NW, _CHUNK), jnp.int32),
            pltpu.VMEM((R // _CHUNK // _NW, _CHUNK), jnp.int32),
            pltpu.VMEM((_CHUNK, _ROW), jnp.float32),
            pltpu.VMEM((_CHUNK, _ROW), jnp.float32),
            pltpu.SemaphoreType.DMA,
        ],
    )(qq, kv, iq, ik)

    G = _NHASH * BH
    W = _BKT_PER_STEP * _BKT
    ol_s = pl.pallas_call(
        _attn_body,
        grid=(G, N // W),
        in_specs=[
            pl.BlockSpec((1, W, _ROW), lambda g, b: (g, b, 0)),
            pl.BlockSpec((1, W, _ROW), lambda g, b: (g, b, 0)),
        ],
        out_specs=pl.BlockSpec((1, W, _ROW), lambda g, b: (g, b, 0)),
        out_shape=jax.ShapeDtypeStruct((G, N, _ROW), jnp.float32),
    )(s_q.reshape(G, N, _ROW), s_kv.reshape(G, N, _ROW))

    o_u = pl.kernel(
        _sc_scatter_body,
        mesh=mesh,
        out_type=jax.ShapeDtypeStruct((R, _ROW), jnp.float32),
        scratch_types=[
            pltpu.VMEM((R // _CHUNK // _NW, _CHUNK), jnp.int32),
            pltpu.VMEM((_CHUNK, _ROW), jnp.float32),
            pltpu.SemaphoreType.DMA,
        ],
    )(ol_s.reshape(R, _ROW), si)

    ou5 = o_u.reshape(_NHASH, B, H, N, _ROW)
    out = pl.pallas_call(
        _combine_body,
        grid=(B, nbkt),
        in_specs=[
            pl.BlockSpec((_NHASH, 1, H, _BKT, _ROW),
                         lambda b, n: (0, b, 0, n, 0)),
            pl.BlockSpec((E, H * E), lambda b, n: (0, 0)),
            pl.BlockSpec((1, E), lambda b, n: (0, 0)),
        ],
        out_specs=pl.BlockSpec((1, _BKT, E), lambda b, n: (b, n, 0)),
        out_shape=jax.ShapeDtypeStruct((B, N, E), jnp.float32),
    )(ou5, W_out, b_out.reshape(1, E))
    return out


# speedup vs baseline: 61.6559x; 1.1435x over previous
import math

import jax
import jax.numpy as jnp
from jax import lax
from jax.experimental.compute_on import compute_on
from jax.experimental import pallas as pl
from jax.experimental.pallas import tpu as pltpu
from jax.experimental.pallas import tpu_sc as plsc

_H = 16
_E = 64
_BKT = 128
_NHASH = 4

_NW = 32
_CHUNK = 128
_ROW = 2 * _E
_BKT_PER_STEP = 8


def _attn_body(q_ref, kv_ref, ol_ref):
    temp = 1.0 / math.sqrt(_E)
    for i in range(_BKT_PER_STEP):
        sl = pl.ds(i * _BKT, _BKT)
        qb = q_ref[0, sl, 0:_E]
        kb = kv_ref[0, sl, 0:_E]
        vb = kv_ref[0, sl, _E:_ROW]
        s = lax.dot_general(qb, kb, (((1,), (1,)), ((), ())),
                            preferred_element_type=jnp.float32) * temp
        m = jnp.max(s, axis=-1, keepdims=True)
        p = jnp.exp(s - m)
        den = jnp.sum(p, axis=-1, keepdims=True)
        o = lax.dot_general(p, vb, (((1,), (0,)), ((), ())),
                            preferred_element_type=jnp.float32)
        lse = jnp.broadcast_to(m + jnp.log(den), (_BKT, _E))
        ol_ref[0, sl, :] = jnp.concatenate([o / den, lse], axis=-1)


def _combine_body(ol_ref, w_ref, b_ref, out_ref):
    acc = jnp.zeros((_BKT, _E), jnp.float32)
    for h in range(_H):
        lh = ol_ref[:, 0, h, :, _E:_E + 1]
        mh = jnp.max(lh, axis=0, keepdims=True)
        wgt = jnp.exp(lh - mh)
        den = jnp.sum(wgt, axis=0)
        oh = ol_ref[:, 0, h, :, 0:_E]
        num = jnp.sum(oh * wgt, axis=0)
        ch = num / den
        wh = w_ref[:, h * _E:(h + 1) * _E]
        acc = acc + lax.dot_general(ch, wh, (((1,), (1,)), ((), ())),
                                    preferred_element_type=jnp.float32)
    out_ref[0] = acc + b_ref[0]


def _sc_gather_body(qq_hbm, kv_hbm, iq_hbm, ik_hbm,
                    sq_hbm, skv_hbm,
                    iq_v, ik_v, bq, bkv, sem):
    wid = lax.axis_index("s") * 2 + lax.axis_index("c")
    nchunk = iq_hbm.shape[0] // _NW
    pltpu.sync_copy(iq_hbm.at[pl.ds(wid * nchunk, nchunk)], iq_v)
    pltpu.sync_copy(ik_hbm.at[pl.ds(wid * nchunk, nchunk)], ik_v)
    for j in range(64):
        row0 = (wid * 64 + j) * _CHUNK
        cq = pltpu.async_copy(qq_hbm.at[iq_v.at[j]], bq, sem)
        ck = pltpu.async_copy(kv_hbm.at[ik_v.at[j]], bkv, sem)
        cq.wait()
        ck.wait()
        pltpu.sync_copy(bq, sq_hbm.at[pl.ds(row0, _CHUNK)])
        pltpu.sync_copy(bkv, skv_hbm.at[pl.ds(row0, _CHUNK)])


def _sc_scatter_body(ol_hbm, si_hbm, ou_hbm, si_v, bo, sem):
    wid = lax.axis_index("s") * 2 + lax.axis_index("c")
    nchunk = si_hbm.shape[0] // _NW
    pltpu.sync_copy(si_hbm.at[pl.ds(wid * nchunk, nchunk)], si_v)
    for j in range(64):
        row0 = (wid * 64 + j) * _CHUNK
        pltpu.sync_copy(ol_hbm.at[pl.ds(row0, _CHUNK)], bo)
        pltpu.async_copy(bo, ou_hbm.at[si_v.at[j]], sem).wait()


def kernel(query, key, value, alpha, beta, W_out, b_out):
    B, N, D = query.shape
    H, E = _H, _E
    BH = B * H
    nbkt = N // _BKT
    R = _NHASH * BH * N

    q = query.reshape(B, N, H, E).transpose(0, 2, 1, 3).reshape(BH, N, E)
    k = key.reshape(B, N, H, E).transpose(0, 2, 1, 3).reshape(BH, N, E)

    q_norms = jnp.linalg.norm(q, axis=-1, keepdims=True)
    k_norms = jnp.linalg.norm(k, axis=-1, keepdims=True)
    MQ = jnp.max(q_norms, axis=-2, keepdims=True)
    MK = jnp.max(k_norms, axis=-2, keepdims=True)
    q_ext = jnp.sqrt(jnp.maximum(MQ ** 2 + MK ** 2 - q_norms ** 2, 0.0))
    k_ext = jnp.sqrt(jnp.maximum(MQ ** 2 + MK ** 2 - k_norms ** 2, 0.0))
    q_t = jnp.concatenate([q, q_ext, jnp.zeros_like(q_ext)], axis=-1)
    k_t = jnp.concatenate([k, jnp.zeros_like(k_ext), k_ext], axis=-1)
    q_hash = jnp.transpose(q_t @ alpha + beta, (2, 0, 1))
    k_hash = jnp.transpose(k_t @ alpha + beta, (2, 0, 1))

    @compute_on("tpu_sparsecore")
    @jax.jit
    def _sc_argsort(x):
        return jnp.argsort(x.reshape(2 * _NHASH * BH, N), axis=-1)

    p_all = _sc_argsort(jnp.concatenate([q_hash, k_hash], axis=0))
    p_all = p_all.reshape(2 * _NHASH, BH, N)
    pq, pk = p_all[:_NHASH], p_all[_NHASH:]

    bh = jnp.arange(BH, dtype=jnp.int32)
    b_of = (bh // H)[None, :, None]
    h_of = (bh % H)[None, :, None]
    iq = ((b_of * N + pq) * H + h_of).reshape(R // _CHUNK, _CHUNK)
    ik = ((b_of * N + pk) * H + h_of).reshape(R // _CHUNK, _CHUNK)
    si = (pq + (jnp.arange(_NHASH * BH, dtype=jnp.int32) * N)
          .reshape(_NHASH, BH, 1)).reshape(R // _CHUNK, _CHUNK)

    q4 = query.reshape(B, N, H, E)
    qq = jnp.concatenate([q4, q4], axis=-1).reshape(B * N * H, _ROW)
    kv = jnp.concatenate([key.reshape(B, N, H, E),
                          value.reshape(B, N, H, E)],
                         axis=-1).reshape(B * N * H, _ROW)

    mesh = plsc.VectorSubcoreMesh(core_axis_name="c", subcore_axis_name="s")
    s_q, s_kv = pl.kernel(
        _sc_gather_body,
        mesh=mesh,
        out_type=[jax.ShapeDtypeStruct((R, _ROW), jnp.float32)] * 2,
        scratch_types=[
            pltpu.VMEM((R // _CHUNK // _NW, _CHUNK), jnp.int32),
            pltpu.VMEM((R // _CHUNK // _NW, _CHUNK), jnp.int32),
            pltpu.VMEM((_CHUNK, _ROW), jnp.float32),
            pltpu.VMEM((_CHUNK, _ROW), jnp.float32),
            pltpu.SemaphoreType.DMA,
        ],
    )(qq, kv, iq, ik)

    G = _NHASH * BH
    W = _BKT_PER_STEP * _BKT
    ol_s = pl.pallas_call(
        _attn_body,
        grid=(G, N // W),
        in_specs=[
            pl.BlockSpec((1, W, _ROW), lambda g, b: (g, b, 0)),
            pl.BlockSpec((1, W, _ROW), lambda g, b: (g, b, 0)),
        ],
        out_specs=pl.BlockSpec((1, W, _ROW), lambda g, b: (g, b, 0)),
        out_shape=jax.ShapeDtypeStruct((G, N, _ROW), jnp.float32),
    )(s_q.reshape(G, N, _ROW), s_kv.reshape(G, N, _ROW))

    o_u = pl.kernel(
        _sc_scatter_body,
        mesh=mesh,
        out_type=jax.ShapeDtypeStruct((R, _ROW), jnp.float32),
        scratch_types=[
            pltpu.VMEM((R // _CHUNK // _NW, _CHUNK), jnp.int32),
            pltpu.VMEM((_CHUNK, _ROW), jnp.float32),
            pltpu.SemaphoreType.DMA,
        ],
    )(ol_s.reshape(R, _ROW), si)

    ou5 = o_u.reshape(_NHASH, B, H, N, _ROW)
    out = pl.pallas_call(
        _combine_body,
        grid=(B, nbkt),
        in_specs=[
            pl.BlockSpec((_NHASH, 1, H, _BKT, _ROW),
                         lambda b, n: (0, b, 0, n, 0)),
            pl.BlockSpec((E, H * E), lambda b, n: (0, 0)),
            pl.BlockSpec((1, E), lambda b, n: (0, 0)),
        ],
        out_specs=pl.BlockSpec((1, _BKT, E), lambda b, n: (b, n, 0)),
        out_shape=jax.ShapeDtypeStruct((B, N, E), jnp.float32),
    )(ou5, W_out, b_out.reshape(1, E))
    return out


# speedup vs baseline: 64.8479x; 1.0518x over previous
import math

import jax
import jax.numpy as jnp
from jax import lax
from jax.experimental.compute_on import compute_on
from jax.experimental import pallas as pl
from jax.experimental.pallas import tpu as pltpu
from jax.experimental.pallas import tpu_sc as plsc

_H = 16
_E = 64
_BKT = 128
_NHASH = 4

_NW = 32
_CHUNK = 128
_ROW = 2 * _E
_BKT_PER_STEP = 8


def _attn_body(q_ref, kv_ref, ol_ref):
    temp = 1.0 / math.sqrt(_E)
    for i in range(_BKT_PER_STEP):
        sl = pl.ds(i * _BKT, _BKT)
        qb = q_ref[0, sl, 0:_E]
        kb = kv_ref[0, sl, 0:_E]
        vb = kv_ref[0, sl, _E:_ROW]
        s = lax.dot_general(qb, kb, (((1,), (1,)), ((), ())),
                            preferred_element_type=jnp.float32) * temp
        m = jnp.max(s, axis=-1, keepdims=True)
        p = jnp.exp(s - m)
        den = jnp.sum(p, axis=-1, keepdims=True)
        o = lax.dot_general(p, vb, (((1,), (0,)), ((), ())),
                            preferred_element_type=jnp.float32)
        lse = jnp.broadcast_to(m + jnp.log(den), (_BKT, _E))
        ol_ref[0, sl, :] = jnp.concatenate([o / den, lse], axis=-1)


def _combine_body(ol_ref, w_ref, b_ref, out_ref):
    acc = jnp.zeros((_BKT, _E), jnp.float32)
    for h in range(_H):
        lh = ol_ref[:, 0, h, :, _E:_E + 1]
        mh = jnp.max(lh, axis=0, keepdims=True)
        wgt = jnp.exp(lh - mh)
        den = jnp.sum(wgt, axis=0)
        oh = ol_ref[:, 0, h, :, 0:_E]
        num = jnp.sum(oh * wgt, axis=0)
        ch = num / den
        wh = w_ref[:, h * _E:(h + 1) * _E]
        acc = acc + lax.dot_general(ch, wh, (((1,), (1,)), ((), ())),
                                    preferred_element_type=jnp.float32)
    out_ref[0] = acc + b_ref[0]


def _sc_gather_body(qq_hbm, kv_hbm, iq_hbm, ik_hbm,
                    sq_hbm, skv_hbm,
                    iq_v, ik_v, bq, bkv, gsem, wsem):
    wid = lax.axis_index("s") * 2 + lax.axis_index("c")
    nchunk = iq_hbm.shape[0] // _NW
    pltpu.sync_copy(iq_hbm.at[pl.ds(wid * nchunk, nchunk)], iq_v)
    pltpu.sync_copy(ik_hbm.at[pl.ds(wid * nchunk, nchunk)], ik_v)

    def gat(j, buf):
        gq = pltpu.async_copy(qq_hbm.at[iq_v.at[j]], bq.at[buf], gsem)
        gk = pltpu.async_copy(kv_hbm.at[ik_v.at[j]], bkv.at[buf], gsem)
        return gq, gk

    def put(j, buf):
        row0 = (wid * nchunk + j) * _CHUNK
        wq = pltpu.async_copy(bq.at[buf], sq_hbm.at[pl.ds(row0, _CHUNK)],
                              wsem)
        wk = pltpu.async_copy(bkv.at[buf], skv_hbm.at[pl.ds(row0, _CHUNK)],
                              wsem)
        return wq, wk

    g = gat(0, 0)
    w = None
    for j in range(64):
        cur = j & 1
        g[0].wait()
        g[1].wait()
        if w is not None:
            w[0].wait()
            w[1].wait()
        if j + 1 < 64:
            g = gat(j + 1, 1 - cur)
        w = put(j, cur)
    w[0].wait()
    w[1].wait()


def _sc_scatter_body(ol_hbm, si_hbm, ou_hbm, si_v, bo, gsem, wsem):
    wid = lax.axis_index("s") * 2 + lax.axis_index("c")
    nchunk = si_hbm.shape[0] // _NW
    pltpu.sync_copy(si_hbm.at[pl.ds(wid * nchunk, nchunk)], si_v)

    def get(j, buf):
        row0 = (wid * nchunk + j) * _CHUNK
        return pltpu.async_copy(ol_hbm.at[pl.ds(row0, _CHUNK)], bo.at[buf],
                                gsem)

    g = get(0, 0)
    w = None
    for j in range(64):
        cur = j & 1
        g.wait()
        if w is not None:
            w.wait()
        if j + 1 < 64:
            g = get(j + 1, 1 - cur)
        w = pltpu.async_copy(bo.at[cur], ou_hbm.at[si_v.at[j]], wsem)
    w.wait()


def kernel(query, key, value, alpha, beta, W_out, b_out):
    B, N, D = query.shape
    H, E = _H, _E
    BH = B * H
    nbkt = N // _BKT
    R = _NHASH * BH * N

    q = query.reshape(B, N, H, E).transpose(0, 2, 1, 3).reshape(BH, N, E)
    k = key.reshape(B, N, H, E).transpose(0, 2, 1, 3).reshape(BH, N, E)

    q_norms = jnp.linalg.norm(q, axis=-1, keepdims=True)
    k_norms = jnp.linalg.norm(k, axis=-1, keepdims=True)
    MQ = jnp.max(q_norms, axis=-2, keepdims=True)
    MK = jnp.max(k_norms, axis=-2, keepdims=True)
    q_ext = jnp.sqrt(jnp.maximum(MQ ** 2 + MK ** 2 - q_norms ** 2, 0.0))
    k_ext = jnp.sqrt(jnp.maximum(MQ ** 2 + MK ** 2 - k_norms ** 2, 0.0))
    q_t = jnp.concatenate([q, q_ext, jnp.zeros_like(q_ext)], axis=-1)
    k_t = jnp.concatenate([k, jnp.zeros_like(k_ext), k_ext], axis=-1)
    q_hash = jnp.transpose(q_t @ alpha + beta, (2, 0, 1))
    k_hash = jnp.transpose(k_t @ alpha + beta, (2, 0, 1))

    @compute_on("tpu_sparsecore")
    @jax.jit
    def _sc_argsort(x):
        return jnp.argsort(x.reshape(2 * _NHASH * BH, N), axis=-1)

    p_all = _sc_argsort(jnp.concatenate([q_hash, k_hash], axis=0))
    p_all = p_all.reshape(2 * _NHASH, BH, N)
    pq, pk = p_all[:_NHASH], p_all[_NHASH:]

    bh = jnp.arange(BH, dtype=jnp.int32)
    b_of = (bh // H)[None, :, None]
    h_of = (bh % H)[None, :, None]
    iq = ((b_of * N + pq) * H + h_of).reshape(R // _CHUNK, _CHUNK)
    ik = ((b_of * N + pk) * H + h_of).reshape(R // _CHUNK, _CHUNK)
    si = (pq + (jnp.arange(_NHASH * BH, dtype=jnp.int32) * N)
          .reshape(_NHASH, BH, 1)).reshape(R // _CHUNK, _CHUNK)

    q4 = query.reshape(B, N, H, E)
    qq = jnp.concatenate([q4, q4], axis=-1).reshape(B * N * H, _ROW)
    kv = jnp.concatenate([key.reshape(B, N, H, E),
                          value.reshape(B, N, H, E)],
                         axis=-1).reshape(B * N * H, _ROW)

    mesh = plsc.VectorSubcoreMesh(core_axis_name="c", subcore_axis_name="s")
    s_q, s_kv = pl.kernel(
        _sc_gather_body,
        mesh=mesh,
        out_type=[jax.ShapeDtypeStruct((R, _ROW), jnp.float32)] * 2,
        scratch_types=[
            pltpu.VMEM((R // _CHUNK // _NW, _CHUNK), jnp.int32),
            pltpu.VMEM((R // _CHUNK // _NW, _CHUNK), jnp.int32),
            pltpu.VMEM((2, _CHUNK, _ROW), jnp.float32),
            pltpu.VMEM((2, _CHUNK, _ROW), jnp.float32),
            pltpu.SemaphoreType.DMA,
            pltpu.SemaphoreType.DMA,
        ],
    )(qq, kv, iq, ik)

    G = _NHASH * BH
    W = _BKT_PER_STEP * _BKT
    ol_s = pl.pallas_call(
        _attn_body,
        grid=(G, N // W),
        in_specs=[
            pl.BlockSpec((1, W, _ROW), lambda g, b: (g, b, 0)),
            pl.BlockSpec((1, W, _ROW), lambda g, b: (g, b, 0)),
        ],
        out_specs=pl.BlockSpec((1, W, _ROW), lambda g, b: (g, b, 0)),
        out_shape=jax.ShapeDtypeStruct((G, N, _ROW), jnp.float32),
    )(s_q.reshape(G, N, _ROW), s_kv.reshape(G, N, _ROW))

    o_u = pl.kernel(
        _sc_scatter_body,
        mesh=mesh,
        out_type=jax.ShapeDtypeStruct((R, _ROW), jnp.float32),
        scratch_types=[
            pltpu.VMEM((R // _CHUNK // _NW, _CHUNK), jnp.int32),
            pltpu.VMEM((2, _CHUNK, _ROW), jnp.float32),
            pltpu.SemaphoreType.DMA,
            pltpu.SemaphoreType.DMA,
        ],
    )(ol_s.reshape(R, _ROW), si)

    ou5 = o_u.reshape(_NHASH, B, H, N, _ROW)
    out = pl.pallas_call(
        _combine_body,
        grid=(B, nbkt),
        in_specs=[
            pl.BlockSpec((_NHASH, 1, H, _BKT, _ROW),
                         lambda b, n: (0, b, 0, n, 0)),
            pl.BlockSpec((E, H * E), lambda b, n: (0, 0)),
            pl.BlockSpec((1, E), lambda b, n: (0, 0)),
        ],
        out_specs=pl.BlockSpec((1, _BKT, E), lambda b, n: (b, n, 0)),
        out_shape=jax.ShapeDtypeStruct((B, N, E), jnp.float32),
    )(ou5, W_out, b_out.reshape(1, E))
    return out


# speedup vs baseline: 75.3959x; 1.1627x over previous
import math

import jax
import jax.numpy as jnp
from jax import lax
from jax.experimental.compute_on import compute_on
from jax.experimental import pallas as pl
from jax.experimental.pallas import tpu as pltpu
from jax.experimental.pallas import tpu_sc as plsc

_H = 16
_E = 64
_BKT = 128
_NHASH = 4

_NW = 32
_CHUNK = 128
_ROW = 2 * _E
_BKT_PER_STEP = 8


def _attn_body(q_ref, kv_ref, ol_ref):
    temp = 1.0 / math.sqrt(_E)
    for i in range(_BKT_PER_STEP):
        sl = pl.ds(i * _BKT, _BKT)
        qb = q_ref[0, sl, 0:_E]
        kb = kv_ref[0, sl, 0:_E]
        vb = kv_ref[0, sl, _E:_ROW]
        s = lax.dot_general(qb, kb, (((1,), (1,)), ((), ())),
                            preferred_element_type=jnp.float32) * temp
        m = jnp.max(s, axis=-1, keepdims=True)
        p = jnp.exp(s - m)
        den = jnp.sum(p, axis=-1, keepdims=True)
        o = lax.dot_general(p, vb, (((1,), (0,)), ((), ())),
                            preferred_element_type=jnp.float32)
        lse = jnp.broadcast_to(m + jnp.log(den), (_BKT, _E))
        ol_ref[0, sl, :] = jnp.concatenate([o / den, lse], axis=-1)


def _combine_body(ol0, ol1, ol2, ol3, w_ref, b_ref, out_ref):
    ols = (ol0, ol1, ol2, ol3)
    acc = jnp.zeros((_BKT, _E), jnp.float32)
    for h in range(_H):
        ls = [r[0, h, :, _E:_E + 1] for r in ols]
        mh = ls[0]
        for lh in ls[1:]:
            mh = jnp.maximum(mh, lh)
        wgts = [jnp.exp(lh - mh) for lh in ls]
        den = wgts[0] + wgts[1] + wgts[2] + wgts[3]
        num = jnp.zeros((_BKT, _E), jnp.float32)
        for r, wgt in zip(ols, wgts):
            num = num + r[0, h, :, 0:_E] * wgt
        ch = num / den
        wh = w_ref[:, h * _E:(h + 1) * _E]
        acc = acc + lax.dot_general(ch, wh, (((1,), (1,)), ((), ())),
                                    preferred_element_type=jnp.float32)
    out_ref[0] = acc + b_ref[0]


def _sc_gather_body(qq_hbm, kv_hbm, iq_hbm, ik_hbm,
                    sq_hbm, skv_hbm,
                    iq_v, ik_v, bq, bkv, gsem, wsem):
    wid = lax.axis_index("s") * 2 + lax.axis_index("c")
    nchunk = iq_hbm.shape[0] // _NW
    pltpu.sync_copy(iq_hbm.at[pl.ds(wid * nchunk, nchunk)], iq_v)
    pltpu.sync_copy(ik_hbm.at[pl.ds(wid * nchunk, nchunk)], ik_v)

    def gat(j, buf):
        gq = pltpu.async_copy(qq_hbm.at[iq_v.at[j]], bq.at[buf], gsem)
        gk = pltpu.async_copy(kv_hbm.at[ik_v.at[j]], bkv.at[buf], gsem)
        return gq, gk

    def put(j, buf):
        row0 = (wid * nchunk + j) * _CHUNK
        wq = pltpu.async_copy(bq.at[buf], sq_hbm.at[pl.ds(row0, _CHUNK)],
                              wsem)
        wk = pltpu.async_copy(bkv.at[buf], skv_hbm.at[pl.ds(row0, _CHUNK)],
                              wsem)
        return wq, wk

    g = gat(0, 0)
    w = None
    for j in range(nchunk):
        cur = j & 1
        g[0].wait()
        g[1].wait()
        if w is not None:
            w[0].wait()
            w[1].wait()
        if j + 1 < nchunk:
            g = gat(j + 1, 1 - cur)
        w = put(j, cur)
    w[0].wait()
    w[1].wait()


def _sc_scatter_body(ol_hbm, si_hbm, ou_hbm, si_v, bo, gsem, wsem):
    wid = lax.axis_index("s") * 2 + lax.axis_index("c")
    nchunk = si_hbm.shape[0] // _NW
    pltpu.sync_copy(si_hbm.at[pl.ds(wid * nchunk, nchunk)], si_v)

    def get(j, buf):
        row0 = (wid * nchunk + j) * _CHUNK
        return pltpu.async_copy(ol_hbm.at[pl.ds(row0, _CHUNK)], bo.at[buf],
                                gsem)

    g = get(0, 0)
    w = None
    for j in range(nchunk):
        cur = j & 1
        g.wait()
        if w is not None:
            w.wait()
        if j + 1 < nchunk:
            g = get(j + 1, 1 - cur)
        w = pltpu.async_copy(bo.at[cur], ou_hbm.at[si_v.at[j]], wsem)
    w.wait()


def kernel(query, key, value, alpha, beta, W_out, b_out):
    B, N, D = query.shape
    H, E = _H, _E
    BH = B * H
    nbkt = N // _BKT
    R = _NHASH * BH * N

    q = query.reshape(B, N, H, E).transpose(0, 2, 1, 3).reshape(BH, N, E)
    k = key.reshape(B, N, H, E).transpose(0, 2, 1, 3).reshape(BH, N, E)

    q_norms = jnp.linalg.norm(q, axis=-1, keepdims=True)
    k_norms = jnp.linalg.norm(k, axis=-1, keepdims=True)
    MQ = jnp.max(q_norms, axis=-2, keepdims=True)
    MK = jnp.max(k_norms, axis=-2, keepdims=True)
    q_ext = jnp.sqrt(jnp.maximum(MQ ** 2 + MK ** 2 - q_norms ** 2, 0.0))
    k_ext = jnp.sqrt(jnp.maximum(MQ ** 2 + MK ** 2 - k_norms ** 2, 0.0))
    q_t = jnp.concatenate([q, q_ext, jnp.zeros_like(q_ext)], axis=-1)
    k_t = jnp.concatenate([k, jnp.zeros_like(k_ext), k_ext], axis=-1)
    q_hash = jnp.transpose(q_t @ alpha + beta, (2, 0, 1))
    k_hash = jnp.transpose(k_t @ alpha + beta, (2, 0, 1))

    @compute_on("tpu_sparsecore")
    @jax.jit
    def _sc_argsort(x):
        return jnp.argsort(x.reshape(2 * _NHASH * BH, N), axis=-1)

    p_all = _sc_argsort(jnp.concatenate([q_hash, k_hash], axis=0))
    p_all = p_all.reshape(2 * _NHASH, BH, N)
    pq, pk = p_all[:_NHASH], p_all[_NHASH:]

    bh = jnp.arange(BH, dtype=jnp.int32)
    b_of = (bh // H)[None, :, None]
    h_of = (bh % H)[None, :, None]
    iq4 = (b_of * N + pq) * H + h_of
    ik4 = (b_of * N + pk) * H + h_of
    si4 = pq + (jnp.arange(BH, dtype=jnp.int32) * N)[None, :, None]

    q4 = query.reshape(B, N, H, E)
    qq = jnp.concatenate([q4, q4], axis=-1).reshape(B * N * H, _ROW)
    kv = jnp.concatenate([key.reshape(B, N, H, E),
                          value.reshape(B, N, H, E)],
                         axis=-1).reshape(B * N * H, _ROW)

    mesh = plsc.VectorSubcoreMesh(core_axis_name="c", subcore_axis_name="s")
    RH = BH * N
    G = BH
    W = _BKT_PER_STEP * _BKT
    ncol = RH // _CHUNK

    o_parts = []
    for a in range(_NHASH):
        s_q, s_kv = pl.kernel(
            _sc_gather_body,
            mesh=mesh,
            out_type=[jax.ShapeDtypeStruct((RH, _ROW), jnp.float32)] * 2,
            scratch_types=[
                pltpu.VMEM((ncol // _NW, _CHUNK), jnp.int32),
                pltpu.VMEM((ncol // _NW, _CHUNK), jnp.int32),
                pltpu.VMEM((2, _CHUNK, _ROW), jnp.float32),
                pltpu.VMEM((2, _CHUNK, _ROW), jnp.float32),
                pltpu.SemaphoreType.DMA,
                pltpu.SemaphoreType.DMA,
            ],
        )(qq, kv, iq4[a].reshape(ncol, _CHUNK), ik4[a].reshape(ncol, _CHUNK))

        ol_s = pl.pallas_call(
            _attn_body,
            grid=(G, N // W),
            in_specs=[
                pl.BlockSpec((1, W, _ROW), lambda g, b: (g, b, 0)),
                pl.BlockSpec((1, W, _ROW), lambda g, b: (g, b, 0)),
            ],
            out_specs=pl.BlockSpec((1, W, _ROW), lambda g, b: (g, b, 0)),
            out_shape=jax.ShapeDtypeStruct((G, N, _ROW), jnp.float32),
        )(s_q.reshape(G, N, _ROW), s_kv.reshape(G, N, _ROW))

        o_u = pl.kernel(
            _sc_scatter_body,
            mesh=mesh,
            out_type=jax.ShapeDtypeStruct((RH, _ROW), jnp.float32),
            scratch_types=[
                pltpu.VMEM((ncol // _NW, _CHUNK), jnp.int32),
                pltpu.VMEM((2, _CHUNK, _ROW), jnp.float32),
                pltpu.SemaphoreType.DMA,
                pltpu.SemaphoreType.DMA,
            ],
        )(ol_s.reshape(RH, _ROW), si4[a].reshape(ncol, _CHUNK))
        o_parts.append(o_u.reshape(B, H, N, _ROW))

    out = pl.pallas_call(
        _combine_body,
        grid=(B, nbkt),
        in_specs=[
            pl.BlockSpec((1, H, _BKT, _ROW), lambda b, n: (b, 0, n, 0)),
            pl.BlockSpec((1, H, _BKT, _ROW), lambda b, n: (b, 0, n, 0)),
            pl.BlockSpec((1, H, _BKT, _ROW), lambda b, n: (b, 0, n, 0)),
            pl.BlockSpec((1, H, _BKT, _ROW), lambda b, n: (b, 0, n, 0)),
            pl.BlockSpec((E, H * E), lambda b, n: (0, 0)),
            pl.BlockSpec((1, E), lambda b, n: (0, 0)),
        ],
        out_specs=pl.BlockSpec((1, _BKT, E), lambda b, n: (b, n, 0)),
        out_shape=jax.ShapeDtypeStruct((B, N, E), jnp.float32),
    )(*o_parts, W_out, b_out.reshape(1, E))
    return out


# speedup vs baseline: 84.4389x; 1.1199x over previous
import math

import jax
import jax.numpy as jnp
from jax import lax
from jax.experimental.compute_on import compute_on
from jax.experimental import pallas as pl
from jax.experimental.pallas import tpu as pltpu
from jax.experimental.pallas import tpu_sc as plsc

_H = 16
_E = 64
_BKT = 128
_NHASH = 4

_NW = 32
_CHUNK = 128
_ROW = 2 * _E
_BKT_PER_STEP = 8


def _attn_body(q_ref, kv_ref, ol_ref):
    temp = 1.0 / math.sqrt(_E)
    for i in range(_BKT_PER_STEP):
        sl = pl.ds(i * _BKT, _BKT)
        qb = q_ref[0, sl, 0:_E]
        kb = kv_ref[0, sl, 0:_E]
        vb = kv_ref[0, sl, _E:_ROW]
        s = lax.dot_general(qb, kb, (((1,), (1,)), ((), ())),
                            preferred_element_type=jnp.float32) * temp
        p = jnp.exp(s)
        den = jnp.sum(p, axis=-1, keepdims=True)
        o = lax.dot_general(p, vb, (((1,), (0,)), ((), ())),
                            preferred_element_type=jnp.float32)
        lse = jnp.broadcast_to(jnp.log(den), (_BKT, _E))
        ol_ref[0, sl, :] = jnp.concatenate([o / den, lse], axis=-1)


def _combine_body(ol0, ol1, ol2, ol3, w_ref, b_ref, out_ref):
    ols = (ol0, ol1, ol2, ol3)
    acc = jnp.zeros((_BKT, _E), jnp.float32)
    for h in range(_H):
        ls = [r[0, h, :, _E:_E + 1] for r in ols]
        mh = ls[0]
        for lh in ls[1:]:
            mh = jnp.maximum(mh, lh)
        wgts = [jnp.exp(lh - mh) for lh in ls]
        den = wgts[0] + wgts[1] + wgts[2] + wgts[3]
        num = jnp.zeros((_BKT, _E), jnp.float32)
        for r, wgt in zip(ols, wgts):
            num = num + r[0, h, :, 0:_E] * wgt
        ch = num / den
        wh = w_ref[:, h * _E:(h + 1) * _E]
        acc = acc + lax.dot_general(ch, wh, (((1,), (1,)), ((), ())),
                                    preferred_element_type=jnp.float32)
    out_ref[0] = acc + b_ref[0]


def _sc_gather_body(qq_hbm, kv_hbm, iq_hbm, ik_hbm,
                    sq_hbm, skv_hbm,
                    iq_v, ik_v, bq, bkv, gsem, wsem):
    wid = lax.axis_index("s") * 2 + lax.axis_index("c")
    nchunk = iq_hbm.shape[0] // _NW
    pltpu.sync_copy(iq_hbm.at[pl.ds(wid * nchunk, nchunk)], iq_v)
    pltpu.sync_copy(ik_hbm.at[pl.ds(wid * nchunk, nchunk)], ik_v)

    def gat(j, buf):
        gq = pltpu.async_copy(qq_hbm.at[iq_v.at[j]], bq.at[buf], gsem)
        gk = pltpu.async_copy(kv_hbm.at[ik_v.at[j]], bkv.at[buf], gsem)
        return gq, gk

    def put(j, buf):
        row0 = (wid * nchunk + j) * _CHUNK
        wq = pltpu.async_copy(bq.at[buf], sq_hbm.at[pl.ds(row0, _CHUNK)],
                              wsem)
        wk = pltpu.async_copy(bkv.at[buf], skv_hbm.at[pl.ds(row0, _CHUNK)],
                              wsem)
        return wq, wk

    g = gat(0, 0)
    w = None
    for j in range(nchunk):
        cur = j & 1
        g[0].wait()
        g[1].wait()
        if w is not None:
            w[0].wait()
            w[1].wait()
        if j + 1 < nchunk:
            g = gat(j + 1, 1 - cur)
        w = put(j, cur)
    w[0].wait()
    w[1].wait()


def _sc_scatter_body(ol_hbm, si_hbm, ou_hbm, si_v, bo, gsem, wsem):
    wid = lax.axis_index("s") * 2 + lax.axis_index("c")
    nchunk = si_hbm.shape[0] // _NW
    pltpu.sync_copy(si_hbm.at[pl.ds(wid * nchunk, nchunk)], si_v)

    def get(j, buf):
        row0 = (wid * nchunk + j) * _CHUNK
        return pltpu.async_copy(ol_hbm.at[pl.ds(row0, _CHUNK)], bo.at[buf],
                                gsem)

    g = get(0, 0)
    w = None
    for j in range(nchunk):
        cur = j & 1
        g.wait()
        if w is not None:
            w.wait()
        if j + 1 < nchunk:
            g = get(j + 1, 1 - cur)
        w = pltpu.async_copy(bo.at[cur], ou_hbm.at[si_v.at[j]], wsem)
    w.wait()


def kernel(query, key, value, alpha, beta, W_out, b_out):
    B, N, D = query.shape
    H, E = _H, _E
    BH = B * H
    nbkt = N // _BKT
    R = _NHASH * BH * N

    q = query.reshape(B, N, H, E).transpose(0, 2, 1, 3).reshape(BH, N, E)
    k = key.reshape(B, N, H, E).transpose(0, 2, 1, 3).reshape(BH, N, E)

    q_norms = jnp.linalg.norm(q, axis=-1, keepdims=True)
    k_norms = jnp.linalg.norm(k, axis=-1, keepdims=True)
    MQ = jnp.max(q_norms, axis=-2, keepdims=True)
    MK = jnp.max(k_norms, axis=-2, keepdims=True)
    q_ext = jnp.sqrt(jnp.maximum(MQ ** 2 + MK ** 2 - q_norms ** 2, 0.0))
    k_ext = jnp.sqrt(jnp.maximum(MQ ** 2 + MK ** 2 - k_norms ** 2, 0.0))
    q_t = jnp.concatenate([q, q_ext, jnp.zeros_like(q_ext)], axis=-1)
    k_t = jnp.concatenate([k, jnp.zeros_like(k_ext), k_ext], axis=-1)
    q_hash = jnp.transpose(q_t @ alpha + beta, (2, 0, 1))
    k_hash = jnp.transpose(k_t @ alpha + beta, (2, 0, 1))

    @compute_on("tpu_sparsecore")
    @jax.jit
    def _sc_argsort(x):
        return jnp.argsort(x.reshape(2 * _NHASH * BH, N), axis=-1)

    p_all = _sc_argsort(jnp.concatenate([q_hash, k_hash], axis=0))
    p_all = p_all.reshape(2 * _NHASH, BH, N)
    pq, pk = p_all[:_NHASH], p_all[_NHASH:]

    bh = jnp.arange(BH, dtype=jnp.int32)
    b_of = (bh // H)[None, :, None]
    h_of = (bh % H)[None, :, None]
    iq4 = (b_of * N + pq) * H + h_of
    ik4 = (b_of * N + pk) * H + h_of
    si4 = pq + (jnp.arange(BH, dtype=jnp.int32) * N)[None, :, None]

    q4 = query.reshape(B, N, H, E)
    qq = jnp.concatenate([q4, q4], axis=-1).reshape(B * N * H, _ROW)
    kv = jnp.concatenate([key.reshape(B, N, H, E),
                          value.reshape(B, N, H, E)],
                         axis=-1).reshape(B * N * H, _ROW)

    mesh = plsc.VectorSubcoreMesh(core_axis_name="c", subcore_axis_name="s")
    RH = BH * N
    G = BH
    W = _BKT_PER_STEP * _BKT
    ncol = RH // _CHUNK

    o_parts = []
    for a in range(_NHASH):
        s_q, s_kv = pl.kernel(
            _sc_gather_body,
            mesh=mesh,
            out_type=[jax.ShapeDtypeStruct((RH, _ROW), jnp.float32)] * 2,
            scratch_types=[
                pltpu.VMEM((ncol // _NW, _CHUNK), jnp.int32),
                pltpu.VMEM((ncol // _NW, _CHUNK), jnp.int32),
                pltpu.VMEM((2, _CHUNK, _ROW), jnp.float32),
                pltpu.VMEM((2, _CHUNK, _ROW), jnp.float32),
                pltpu.SemaphoreType.DMA,
                pltpu.SemaphoreType.DMA,
            ],
        )(qq, kv, iq4[a].reshape(ncol, _CHUNK), ik4[a].reshape(ncol, _CHUNK))

        ol_s = pl.pallas_call(
            _attn_body,
            grid=(G, N // W),
            in_specs=[
                pl.BlockSpec((1, W, _ROW), lambda g, b: (g, b, 0)),
                pl.BlockSpec((1, W, _ROW), lambda g, b: (g, b, 0)),
            ],
            out_specs=pl.BlockSpec((1, W, _ROW), lambda g, b: (g, b, 0)),
            out_shape=jax.ShapeDtypeStruct((G, N, _ROW), jnp.float32),
        )(s_q.reshape(G, N, _ROW), s_kv.reshape(G, N, _ROW))

        o_u = pl.kernel(
            _sc_scatter_body,
            mesh=mesh,
            out_type=jax.ShapeDtypeStruct((RH, _ROW), jnp.float32),
            scratch_types=[
                pltpu.VMEM((ncol // _NW, _CHUNK), jnp.int32),
                pltpu.VMEM((2, _CHUNK, _ROW), jnp.float32),
                pltpu.SemaphoreType.DMA,
                pltpu.SemaphoreType.DMA,
            ],
        )(ol_s.reshape(RH, _ROW), si4[a].reshape(ncol, _CHUNK))
        o_parts.append(o_u.reshape(B, H, N, _ROW))

    out = pl.pallas_call(
        _combine_body,
        grid=(B, nbkt),
        in_specs=[
            pl.BlockSpec((1, H, _BKT, _ROW), lambda b, n: (b, 0, n, 0)),
            pl.BlockSpec((1, H, _BKT, _ROW), lambda b, n: (b, 0, n, 0)),
            pl.BlockSpec((1, H, _BKT, _ROW), lambda b, n: (b, 0, n, 0)),
            pl.BlockSpec((1, H, _BKT, _ROW), lambda b, n: (b, 0, n, 0)),
            pl.BlockSpec((E, H * E), lambda b, n: (0, 0)),
            pl.BlockSpec((1, E), lambda b, n: (0, 0)),
        ],
        out_specs=pl.BlockSpec((1, _BKT, E), lambda b, n: (b, n, 0)),
        out_shape=jax.ShapeDtypeStruct((B, N, E), jnp.float32),
    )(*o_parts, W_out, b_out.reshape(1, E))
    return out


# speedup vs baseline: 89.0129x; 1.0542x over previous
import math

import jax
import jax.numpy as jnp
from jax import lax
from jax.experimental.compute_on import compute_on
from jax.experimental import pallas as pl
from jax.experimental.pallas import tpu as pltpu
from jax.experimental.pallas import tpu_sc as plsc

_H = 16
_E = 64
_BKT = 128
_NHASH = 4

_NW = 32
_CHUNK = 128
_ROW = 2 * _E
_BKT_PER_STEP = 16


def _attn_body(q_ref, kv_ref, ol_ref):
    temp = 1.0 / math.sqrt(_E)
    for i in range(_BKT_PER_STEP):
        sl = pl.ds(i * _BKT, _BKT)
        qb = q_ref[0, sl, 0:_E]
        kb = kv_ref[0, sl, 0:_E]
        vb = kv_ref[0, sl, _E:_ROW]
        s = lax.dot_general(qb, kb, (((1,), (1,)), ((), ())),
                            preferred_element_type=jnp.float32) * temp
        p = jnp.exp(s)
        den = jnp.sum(p, axis=-1, keepdims=True)
        o = lax.dot_general(p, vb, (((1,), (0,)), ((), ())),
                            preferred_element_type=jnp.float32)
        lse = jnp.broadcast_to(jnp.log(den), (_BKT, _E))
        ol_ref[0, sl, :] = jnp.concatenate([o / den, lse], axis=-1)


def _combine_body(ol0, ol1, ol2, ol3, w_ref, b_ref, out_ref):
    ols = (ol0, ol1, ol2, ol3)
    acc = jnp.zeros((_BKT, _E), jnp.float32)
    for h in range(_H):
        ls = [r[0, h, :, _E:_E + 1] for r in ols]
        mh = ls[0]
        for lh in ls[1:]:
            mh = jnp.maximum(mh, lh)
        wgts = [jnp.exp(lh - mh) for lh in ls]
        den = wgts[0] + wgts[1] + wgts[2] + wgts[3]
        num = jnp.zeros((_BKT, _E), jnp.float32)
        for r, wgt in zip(ols, wgts):
            num = num + r[0, h, :, 0:_E] * wgt
        ch = num / den
        wh = w_ref[:, h * _E:(h + 1) * _E]
        acc = acc + lax.dot_general(ch, wh, (((1,), (1,)), ((), ())),
                                    preferred_element_type=jnp.float32)
    out_ref[0] = acc + b_ref[0]


def _sc_gather_body(qq_hbm, kv_hbm, iq_hbm, ik_hbm,
                    sq_hbm, skv_hbm,
                    iq_v, ik_v, bq, bkv, gsem, wsem):
    wid = lax.axis_index("s") * 2 + lax.axis_index("c")
    nchunk = iq_hbm.shape[0] // _NW
    pltpu.sync_copy(iq_hbm.at[pl.ds(wid * nchunk, nchunk)], iq_v)
    pltpu.sync_copy(ik_hbm.at[pl.ds(wid * nchunk, nchunk)], ik_v)

    def gat(j, buf):
        gq = pltpu.async_copy(qq_hbm.at[iq_v.at[j]], bq.at[buf], gsem)
        gk = pltpu.async_copy(kv_hbm.at[ik_v.at[j]], bkv.at[buf], gsem)
        return gq, gk

    def put(j, buf):
        row0 = (wid * nchunk + j) * _CHUNK
        wq = pltpu.async_copy(bq.at[buf], sq_hbm.at[pl.ds(row0, _CHUNK)],
                              wsem)
        wk = pltpu.async_copy(bkv.at[buf], skv_hbm.at[pl.ds(row0, _CHUNK)],
                              wsem)
        return wq, wk

    g = gat(0, 0)
    w = None
    for j in range(nchunk):
        cur = j & 1
        g[0].wait()
        g[1].wait()
        if w is not None:
            w[0].wait()
            w[1].wait()
        if j + 1 < nchunk:
            g = gat(j + 1, 1 - cur)
        w = put(j, cur)
    w[0].wait()
    w[1].wait()


def _sc_scatter_body(ol_hbm, si_hbm, ou_hbm, si_v, bo, gsem, wsem):
    wid = lax.axis_index("s") * 2 + lax.axis_index("c")
    nchunk = si_hbm.shape[0] // _NW
    pltpu.sync_copy(si_hbm.at[pl.ds(wid * nchunk, nchunk)], si_v)

    def get(j, buf):
        row0 = (wid * nchunk + j) * _CHUNK
        return pltpu.async_copy(ol_hbm.at[pl.ds(row0, _CHUNK)], bo.at[buf],
                                gsem)

    g = get(0, 0)
    w = None
    for j in range(nchunk):
        cur = j & 1
        g.wait()
        if w is not None:
            w.wait()
        if j + 1 < nchunk:
            g = get(j + 1, 1 - cur)
        w = pltpu.async_copy(bo.at[cur], ou_hbm.at[si_v.at[j]], wsem)
    w.wait()


def kernel(query, key, value, alpha, beta, W_out, b_out):
    B, N, D = query.shape
    H, E = _H, _E
    BH = B * H
    nbkt = N // _BKT
    R = _NHASH * BH * N

    q = query.reshape(B, N, H, E).transpose(0, 2, 1, 3).reshape(BH, N, E)
    k = key.reshape(B, N, H, E).transpose(0, 2, 1, 3).reshape(BH, N, E)

    q_norms = jnp.linalg.norm(q, axis=-1, keepdims=True)
    k_norms = jnp.linalg.norm(k, axis=-1, keepdims=True)
    MQ = jnp.max(q_norms, axis=-2, keepdims=True)
    MK = jnp.max(k_norms, axis=-2, keepdims=True)
    q_ext = jnp.sqrt(jnp.maximum(MQ ** 2 + MK ** 2 - q_norms ** 2, 0.0))
    k_ext = jnp.sqrt(jnp.maximum(MQ ** 2 + MK ** 2 - k_norms ** 2, 0.0))
    q_t = jnp.concatenate([q, q_ext, jnp.zeros_like(q_ext)], axis=-1)
    k_t = jnp.concatenate([k, jnp.zeros_like(k_ext), k_ext], axis=-1)
    q_hash = jnp.transpose(q_t @ alpha + beta, (2, 0, 1))
    k_hash = jnp.transpose(k_t @ alpha + beta, (2, 0, 1))

    @compute_on("tpu_sparsecore")
    @jax.jit
    def _sc_argsort(x):
        return jnp.argsort(x.reshape(2 * _NHASH * BH, N), axis=-1)

    p_all = _sc_argsort(jnp.concatenate([q_hash, k_hash], axis=0))
    p_all = p_all.reshape(2 * _NHASH, BH, N)
    pq, pk = p_all[:_NHASH], p_all[_NHASH:]

    bh = jnp.arange(BH, dtype=jnp.int32)
    b_of = (bh // H)[None, :, None]
    h_of = (bh % H)[None, :, None]
    iq4 = (b_of * N + pq) * H + h_of
    ik4 = (b_of * N + pk) * H + h_of
    si4 = pq + (jnp.arange(BH, dtype=jnp.int32) * N)[None, :, None]

    q4 = query.reshape(B, N, H, E)
    qq = jnp.concatenate([q4, q4], axis=-1).reshape(B * N * H, _ROW)
    kv = jnp.concatenate([key.reshape(B, N, H, E),
                          value.reshape(B, N, H, E)],
                         axis=-1).reshape(B * N * H, _ROW)

    mesh = plsc.VectorSubcoreMesh(core_axis_name="c", subcore_axis_name="s")
    RH = BH * N
    G = BH
    W = _BKT_PER_STEP * _BKT
    ncol = RH // _CHUNK

    o_parts = []
    for a in range(_NHASH):
        s_q, s_kv = pl.kernel(
            _sc_gather_body,
            mesh=mesh,
            out_type=[jax.ShapeDtypeStruct((RH, _ROW), jnp.float32)] * 2,
            scratch_types=[
                pltpu.VMEM((ncol // _NW, _CHUNK), jnp.int32),
                pltpu.VMEM((ncol // _NW, _CHUNK), jnp.int32),
                pltpu.VMEM((2, _CHUNK, _ROW), jnp.float32),
                pltpu.VMEM((2, _CHUNK, _ROW), jnp.float32),
                pltpu.SemaphoreType.DMA,
                pltpu.SemaphoreType.DMA,
            ],
        )(qq, kv, iq4[a].reshape(ncol, _CHUNK), ik4[a].reshape(ncol, _CHUNK))

        ol_s = pl.pallas_call(
            _attn_body,
            grid=(G, N // W),
            in_specs=[
                pl.BlockSpec((1, W, _ROW), lambda g, b: (g, b, 0)),
                pl.BlockSpec((1, W, _ROW), lambda g, b: (g, b, 0)),
            ],
            out_specs=pl.BlockSpec((1, W, _ROW), lambda g, b: (g, b, 0)),
            out_shape=jax.ShapeDtypeStruct((G, N, _ROW), jnp.float32),
        )(s_q.reshape(G, N, _ROW), s_kv.reshape(G, N, _ROW))

        o_u = pl.kernel(
            _sc_scatter_body,
            mesh=mesh,
            out_type=jax.ShapeDtypeStruct((RH, _ROW), jnp.float32),
            scratch_types=[
                pltpu.VMEM((ncol // _NW, _CHUNK), jnp.int32),
                pltpu.VMEM((2, _CHUNK, _ROW), jnp.float32),
                pltpu.SemaphoreType.DMA,
                pltpu.SemaphoreType.DMA,
            ],
        )(ol_s.reshape(RH, _ROW), si4[a].reshape(ncol, _CHUNK))
        o_parts.append(o_u.reshape(B, H, N, _ROW))

    out = pl.pallas_call(
        _combine_body,
        grid=(B, nbkt),
        in_specs=[
            pl.BlockSpec((1, H, _BKT, _ROW), lambda b, n: (b, 0, n, 0)),
            pl.BlockSpec((1, H, _BKT, _ROW), lambda b, n: (b, 0, n, 0)),
            pl.BlockSpec((1, H, _BKT, _ROW), lambda b, n: (b, 0, n, 0)),
            pl.BlockSpec((1, H, _BKT, _ROW), lambda b, n: (b, 0, n, 0)),
            pl.BlockSpec((E, H * E), lambda b, n: (0, 0)),
            pl.BlockSpec((1, E), lambda b, n: (0, 0)),
        ],
        out_specs=pl.BlockSpec((1, _BKT, E), lambda b, n: (b, n, 0)),
        out_shape=jax.ShapeDtypeStruct((B, N, E), jnp.float32),
    )(*o_parts, W_out, b_out.reshape(1, E))
    return out
